# baseline (speedup 1.0000x reference)
import functools

import jax
import jax.numpy as jnp
from jax.experimental import pallas as pl
from jax.experimental.pallas import tpu as pltpu

_MIB = 1024 * 1024


def _fused_kernel(q_ref, k_ref, v_ref, wqh_ref, wkh_ref, wf_ref,
                  out_ref, attn_ref, *, inv_heads):
    v = v_ref[...].astype(jnp.bfloat16)
    out_ref[...] = jnp.dot(v, wf_ref[...], preferred_element_type=jnp.float32)

    q = q_ref[...].astype(jnp.bfloat16)
    k = k_ref[...].astype(jnp.bfloat16)
    qh = jnp.dot(q, wqh_ref[...], preferred_element_type=jnp.float32)
    kh = jnp.dot(k, wkh_ref[...], preferred_element_type=jnp.float32)
    s = jax.lax.dot_general(qh.astype(jnp.bfloat16), kh.astype(jnp.bfloat16),
                            (((1,), (1,)), ((), ())),
                            preferred_element_type=jnp.float32)
    s = s - jnp.max(s, axis=-1, keepdims=True)
    e = jnp.exp(s)
    attn_ref[...] = e * (inv_heads / jnp.sum(e, axis=-1, keepdims=True))


def kernel(key, value, query, wq, wk, wv, wo):
    num_heads = 8
    B, Lk, Dk = key.shape
    _, Lv, Dv = value.shape
    _, Lq, _ = query.shape
    Dout = wo.shape[0]
    head_dim = Dk // num_heads
    lo = (num_heads - 1) * head_dim
    scale = head_dim ** (-0.5)

    hp = max(128, head_dim)
    wqh = jnp.zeros((Dk, hp), jnp.bfloat16).at[:, :head_dim].set(
        (scale * wq[lo:lo + head_dim, :]).T.astype(jnp.bfloat16))
    wkh = jnp.zeros((Dk, hp), jnp.bfloat16).at[:, :head_dim].set(
        wk[lo:lo + head_dim, :].T.astype(jnp.bfloat16))
    w_fused = (wv.T @ wo.T).astype(jnp.bfloat16)

    kern = functools.partial(_fused_kernel, inv_heads=1.0 / num_heads)

    tq = 256 if Lq % 256 == 0 else Lq
    grid_i = Lq // tq

    in_bytes = 4 * (tq * Dk + Lk * Dk + tq * Dv)
    out_bytes = 4 * (tq * Dout + tq * Lk)
    w_bytes = 2 * (2 * Dk * hp + Dv * Dout)
    vmem = 2 * (in_bytes + out_bytes) + w_bytes + 6 * tq * Lk * 4

    cost = pl.CostEstimate(
        flops=2 * B * (Lv * Dv * Dout + (Lq + Lk) * Dk * hp + Lq * Lk * hp),
        transcendentals=B * Lq * Lk,
        bytes_accessed=B * 4 * (Lq * Dk + Lk * Dk + Lv * Dv
                                + Lv * Dout + Lq * Lk) + w_bytes)

    out, attn = pl.pallas_call(
        kern,
        out_shape=(jax.ShapeDtypeStruct((B, Lv, Dout), jnp.float32),
                   jax.ShapeDtypeStruct((B, Lq, Lk), jnp.float32)),
        grid=(B, grid_i),
        in_specs=[
            pl.BlockSpec((None, tq, Dk), lambda b, i: (b, i, 0)),
            pl.BlockSpec((None, Lk, Dk), lambda b, i: (b, 0, 0)),
            pl.BlockSpec((None, tq, Dv), lambda b, i: (b, i, 0)),
            pl.BlockSpec((Dk, hp), lambda b, i: (0, 0)),
            pl.BlockSpec((Dk, hp), lambda b, i: (0, 0)),
            pl.BlockSpec((Dv, Dout), lambda b, i: (0, 0)),
        ],
        out_specs=(pl.BlockSpec((None, tq, Dout), lambda b, i: (b, i, 0)),
                   pl.BlockSpec((None, tq, Lk), lambda b, i: (b, i, 0))),
        compiler_params=pltpu.CompilerParams(
            dimension_semantics=("parallel", "parallel"),
            vmem_limit_bytes=int(min(max(vmem, 32 * _MIB), 64 * _MIB))),
        cost_estimate=cost,
    )(query, key, value, wqh, wkh, w_fused)
    return out, attn


# speedup vs baseline: 1.0012x; 1.0012x over previous
import functools

import jax
import jax.numpy as jnp
from jax.experimental import pallas as pl
from jax.experimental.pallas import tpu as pltpu

_MIB = 1024 * 1024


def _fused_kernel(q_ref, k_ref, v_ref, wqh_ref, wkh_ref, wf_ref,
                  out_ref, attn_ref, *, inv_heads):
    v = v_ref[...].astype(jnp.bfloat16)
    out_ref[...] = jnp.dot(v, wf_ref[...], preferred_element_type=jnp.float32)

    q = q_ref[...].astype(jnp.bfloat16)
    k = k_ref[...].astype(jnp.bfloat16)
    qh = jnp.dot(q, wqh_ref[...], preferred_element_type=jnp.float32)
    kh = jnp.dot(k, wkh_ref[...], preferred_element_type=jnp.float32)
    s = jax.lax.dot_general(qh.astype(jnp.bfloat16), kh.astype(jnp.bfloat16),
                            (((1,), (1,)), ((), ())),
                            preferred_element_type=jnp.float32)
    s = s - jnp.max(s, axis=-1, keepdims=True)
    e = jnp.exp(s)
    attn_ref[...] = e * (inv_heads / jnp.sum(e, axis=-1, keepdims=True))


def kernel(key, value, query, wq, wk, wv, wo):
    num_heads = 8
    B, Lk, Dk = key.shape
    _, Lv, Dv = value.shape
    _, Lq, _ = query.shape
    Dout = wo.shape[0]
    head_dim = Dk // num_heads
    lo = (num_heads - 1) * head_dim
    scale = head_dim ** (-0.5)

    hp = max(128, head_dim)
    wqh = jnp.zeros((Dk, hp), jnp.bfloat16).at[:, :head_dim].set(
        (scale * wq[lo:lo + head_dim, :]).T.astype(jnp.bfloat16))
    wkh = jnp.zeros((Dk, hp), jnp.bfloat16).at[:, :head_dim].set(
        wk[lo:lo + head_dim, :].T.astype(jnp.bfloat16))
    w_fused = (wv.T @ wo.T).astype(jnp.bfloat16)

    kern = functools.partial(_fused_kernel, inv_heads=1.0 / num_heads)

    tq = 256 if Lq % 256 == 0 else Lq
    grid_i = Lq // tq

    in_bytes = 4 * (tq * Dk + Lk * Dk + tq * Dv)
    out_bytes = 4 * (tq * Dout + tq * Lk)
    w_bytes = 2 * (2 * Dk * hp + Dv * Dout)
    vmem = 2 * (in_bytes + out_bytes) + w_bytes + 6 * tq * Lk * 4

    cost = pl.CostEstimate(
        flops=2 * B * (Lv * Dv * Dout + (Lq + Lk) * Dk * hp + Lq * Lk * hp),
        transcendentals=B * Lq * Lk,
        bytes_accessed=B * 4 * (Lq * Dk + Lk * Dk + Lv * Dv
                                + Lv * Dout + Lq * Lk) + w_bytes)

    out, attn = pl.pallas_call(
        kern,
        out_shape=(jax.ShapeDtypeStruct((B, Lv, Dout), jnp.float32),
                   jax.ShapeDtypeStruct((B, Lq, Lk), jnp.float32)),
        grid=(B, grid_i),
        in_specs=[
            pl.BlockSpec((None, tq, Dk), lambda b, i: (b, i, 0)),
            pl.BlockSpec((None, Lk, Dk), lambda b, i: (b, 0, 0)),
            pl.BlockSpec((None, tq, Dv), lambda b, i: (b, i, 0)),
            pl.BlockSpec((Dk, hp), lambda b, i: (0, 0)),
            pl.BlockSpec((Dk, hp), lambda b, i: (0, 0)),
            pl.BlockSpec((Dv, Dout), lambda b, i: (0, 0)),
        ],
        out_specs=(pl.BlockSpec((None, tq, Dout), lambda b, i: (b, i, 0)),
                   pl.BlockSpec((None, tq, Lk), lambda b, i: (b, i, 0))),
        compiler_params=pltpu.CompilerParams(
            dimension_semantics=("parallel", "arbitrary"),
            vmem_limit_bytes=int(min(max(vmem, 32 * _MIB), 64 * _MIB))),
        cost_estimate=cost,
    )(query, key, value, wqh, wkh, w_fused)
    return out, attn


# speedup vs baseline: 1.4974x; 1.4957x over previous
import functools

import jax
import jax.numpy as jnp
from jax.experimental import pallas as pl
from jax.experimental.pallas import tpu as pltpu

_MIB = 1024 * 1024


def _prep_kernel(wo_ref, wv_ref, wf2_ref):
    wf2_ref[...] = jnp.dot(wo_ref[...].astype(jnp.bfloat16),
                           wv_ref[...].astype(jnp.bfloat16),
                           preferred_element_type=jnp.float32
                           ).astype(jnp.bfloat16)


def _fused_kernel(q_ref, k_ref, v_ref, wq_ref, wk_ref, wf2_ref,
                  out_ref, attn_ref, *, lo, head_dim, scale, inv_heads):
    tb = (((1,), (1,)), ((), ()))
    v = v_ref[...].astype(jnp.bfloat16)
    out_ref[...] = jax.lax.dot_general(v, wf2_ref[...], tb,
                                       preferred_element_type=jnp.float32)

    wqh = (wq_ref[lo:lo + head_dim, :].astype(jnp.bfloat16)
           * jnp.bfloat16(scale))
    wkh = wk_ref[lo:lo + head_dim, :].astype(jnp.bfloat16)
    q = q_ref[...].astype(jnp.bfloat16)
    k = k_ref[...].astype(jnp.bfloat16)
    qh = jax.lax.dot_general(q, wqh, tb, preferred_element_type=jnp.float32)
    kh = jax.lax.dot_general(k, wkh, tb, preferred_element_type=jnp.float32)
    s = jax.lax.dot_general(qh.astype(jnp.bfloat16), kh.astype(jnp.bfloat16),
                            tb, preferred_element_type=jnp.float32)
    s = s - jnp.max(s, axis=-1, keepdims=True)
    e = jnp.exp(s)
    attn_ref[...] = e * (inv_heads / jnp.sum(e, axis=-1, keepdims=True))


def kernel(key, value, query, wq, wk, wv, wo):
    num_heads = 8
    B, Lk, Dk = key.shape
    _, Lv, Dv = value.shape
    _, Lq, _ = query.shape
    Dout = wo.shape[0]
    head_dim = Dk // num_heads
    lo = (num_heads - 1) * head_dim
    scale = head_dim ** (-0.5)

    wf2 = pl.pallas_call(
        _prep_kernel,
        out_shape=jax.ShapeDtypeStruct((Dout, Dv), jnp.bfloat16),
        compiler_params=pltpu.CompilerParams(
            vmem_limit_bytes=32 * _MIB),
    )(wo, wv)

    kern = functools.partial(_fused_kernel, lo=lo, head_dim=head_dim,
                             scale=scale, inv_heads=1.0 / num_heads)

    in_bytes = 4 * (Lq * Dk + Lk * Dk + Lv * Dv)
    out_bytes = 4 * (Lv * Dout + Lq * Lk)
    w_bytes = 4 * 2 * Dk * Dk + 2 * Dout * Dv
    vmem = 2 * (in_bytes + out_bytes) + w_bytes + 8 * Lq * Lk * 4

    cost = pl.CostEstimate(
        flops=2 * B * (Lv * Dv * Dout + (Lq + Lk) * Dk * 128 + Lq * Lk * 128),
        transcendentals=B * Lq * Lk,
        bytes_accessed=B * (in_bytes + out_bytes) + w_bytes)

    out, attn = pl.pallas_call(
        kern,
        out_shape=(jax.ShapeDtypeStruct((B, Lv, Dout), jnp.float32),
                   jax.ShapeDtypeStruct((B, Lq, Lk), jnp.float32)),
        grid=(B,),
        in_specs=[
            pl.BlockSpec((None, Lq, Dk), lambda b: (b, 0, 0)),
            pl.BlockSpec((None, Lk, Dk), lambda b: (b, 0, 0)),
            pl.BlockSpec((None, Lv, Dv), lambda b: (b, 0, 0)),
            pl.BlockSpec((Dk, Dk), lambda b: (0, 0)),
            pl.BlockSpec((Dk, Dk), lambda b: (0, 0)),
            pl.BlockSpec((Dout, Dv), lambda b: (0, 0)),
        ],
        out_specs=(pl.BlockSpec((None, Lv, Dout), lambda b: (b, 0, 0)),
                   pl.BlockSpec((None, Lq, Lk), lambda b: (b, 0, 0))),
        compiler_params=pltpu.CompilerParams(
            dimension_semantics=("parallel",),
            vmem_limit_bytes=int(min(max(vmem, 32 * _MIB), 64 * _MIB))),
        cost_estimate=cost,
    )(query, key, value, wq, wk, wf2)
    return out, attn


# speedup vs baseline: 1.7081x; 1.1407x over previous
import functools

import jax
import jax.numpy as jnp
from jax.experimental import pallas as pl
from jax.experimental.pallas import tpu as pltpu

_MIB = 1024 * 1024


def _prep_kernel(wo_ref, wv_ref, wf2_ref):
    wf2_ref[...] = jnp.dot(wo_ref[...].astype(jnp.bfloat16),
                           wv_ref[...].astype(jnp.bfloat16),
                           preferred_element_type=jnp.float32
                           ).astype(jnp.bfloat16)


def _fused_kernel(q_ref, k_ref, v_ref, wq_ref, wk_ref, wf2_ref,
                  out_ref, attn_ref, *, lo, head_dim, scale, inv_heads, nb):
    tb = (((1,), (1,)), ((), ()))
    nL, D = q_ref.shape[0] * q_ref.shape[1], q_ref.shape[2]
    L = q_ref.shape[1]
    v = v_ref[...].astype(jnp.bfloat16).reshape(nL, D)
    out = jax.lax.dot_general(v, wf2_ref[...], tb,
                              preferred_element_type=jnp.float32)
    out_ref[...] = out.reshape(nb, L, out.shape[-1])

    wqh = (wq_ref[lo:lo + head_dim, :].astype(jnp.bfloat16)
           * jnp.bfloat16(scale))
    wkh = wk_ref[lo:lo + head_dim, :].astype(jnp.bfloat16)
    q = q_ref[...].astype(jnp.bfloat16).reshape(nL, D)
    k = k_ref[...].astype(jnp.bfloat16).reshape(nL, D)
    qh = jax.lax.dot_general(q, wqh, tb, preferred_element_type=jnp.float32)
    kh = jax.lax.dot_general(k, wkh, tb, preferred_element_type=jnp.float32)
    qh = qh.astype(jnp.bfloat16).reshape(nb, L, head_dim)
    kh = kh.astype(jnp.bfloat16).reshape(nb, L, head_dim)
    for j in range(nb):
        s = jax.lax.dot_general(qh[j], kh[j], tb,
                                preferred_element_type=jnp.float32)
        s = s - jnp.max(s, axis=-1, keepdims=True)
        e = jnp.exp(s)
        attn_ref[j] = e * (inv_heads / jnp.sum(e, axis=-1, keepdims=True))


def kernel(key, value, query, wq, wk, wv, wo):
    num_heads = 8
    B, Lk, Dk = key.shape
    _, Lv, Dv = value.shape
    _, Lq, _ = query.shape
    Dout = wo.shape[0]
    head_dim = Dk // num_heads
    lo = (num_heads - 1) * head_dim
    scale = head_dim ** (-0.5)

    wf2 = pl.pallas_call(
        _prep_kernel,
        out_shape=jax.ShapeDtypeStruct((Dout, Dv), jnp.bfloat16),
        compiler_params=pltpu.CompilerParams(
            vmem_limit_bytes=32 * _MIB),
    )(wo, wv)

    nb = 2 if B % 2 == 0 else 1
    grid_b = B // nb

    kern = functools.partial(_fused_kernel, lo=lo, head_dim=head_dim,
                             scale=scale, inv_heads=1.0 / num_heads, nb=nb)

    in_bytes = nb * 4 * (Lq * Dk + Lk * Dk + Lv * Dv)
    out_bytes = nb * 4 * (Lv * Dout + Lq * Lk)
    w_bytes = 4 * 2 * Dk * Dk + 2 * Dout * Dv
    vmem = 2 * (in_bytes + out_bytes) + w_bytes + 8 * nb * Lq * Lk * 4

    cost = pl.CostEstimate(
        flops=2 * B * (Lv * Dv * Dout + (Lq + Lk) * Dk * 128 + Lq * Lk * 128),
        transcendentals=B * Lq * Lk,
        bytes_accessed=grid_b * (in_bytes + out_bytes) + w_bytes)

    out, attn = pl.pallas_call(
        kern,
        out_shape=(jax.ShapeDtypeStruct((B, Lv, Dout), jnp.float32),
                   jax.ShapeDtypeStruct((B, Lq, Lk), jnp.float32)),
        grid=(grid_b,),
        in_specs=[
            pl.BlockSpec((nb, Lq, Dk), lambda b: (b, 0, 0)),
            pl.BlockSpec((nb, Lk, Dk), lambda b: (b, 0, 0)),
            pl.BlockSpec((nb, Lv, Dv), lambda b: (b, 0, 0)),
            pl.BlockSpec((Dk, Dk), lambda b: (0, 0)),
            pl.BlockSpec((Dk, Dk), lambda b: (0, 0)),
            pl.BlockSpec((Dout, Dv), lambda b: (0, 0)),
        ],
        out_specs=(pl.BlockSpec((nb, Lv, Dout), lambda b: (b, 0, 0)),
                   pl.BlockSpec((nb, Lq, Lk), lambda b: (b, 0, 0))),
        compiler_params=pltpu.CompilerParams(
            dimension_semantics=("parallel",),
            vmem_limit_bytes=int(min(max(vmem, 32 * _MIB), 64 * _MIB))),
        cost_estimate=cost,
    )(query, key, value, wq, wk, wf2)
    return out, attn


# speedup vs baseline: 1.7804x; 1.0424x over previous
import functools

import jax
import jax.numpy as jnp
from jax.experimental import pallas as pl
from jax.experimental.pallas import tpu as pltpu

_MIB = 1024 * 1024


def _prep_kernel(wo_ref, wv_ref, wf2_ref):
    wf2_ref[...] = jnp.dot(wo_ref[...].astype(jnp.bfloat16),
                           wv_ref[...].astype(jnp.bfloat16),
                           preferred_element_type=jnp.float32
                           ).astype(jnp.bfloat16)


def _fused_kernel(q_ref, k_ref, v_ref, wq_ref, wk_ref, wf2_ref,
                  out_ref, attn_ref, *, lo, head_dim, scale, inv_heads, nb):
    tb = (((1,), (1,)), ((), ()))
    nL, D = q_ref.shape[0] * q_ref.shape[1], q_ref.shape[2]
    L = q_ref.shape[1]
    v = v_ref[...].astype(jnp.bfloat16).reshape(nL, D)
    out = jax.lax.dot_general(v, wf2_ref[...], tb,
                              preferred_element_type=jnp.float32)
    out_ref[...] = out.reshape(nb, L, out.shape[-1])

    wqh = (wq_ref[lo:lo + head_dim, :].astype(jnp.bfloat16)
           * jnp.bfloat16(scale))
    wkh = wk_ref[lo:lo + head_dim, :].astype(jnp.bfloat16)
    q = q_ref[...].astype(jnp.bfloat16).reshape(nL, D)
    k = k_ref[...].astype(jnp.bfloat16).reshape(nL, D)
    qh = jax.lax.dot_general(q, wqh, tb, preferred_element_type=jnp.float32)
    kh = jax.lax.dot_general(k, wkh, tb, preferred_element_type=jnp.float32)
    qh = qh.astype(jnp.bfloat16).reshape(nb, L, head_dim)
    kh = kh.astype(jnp.bfloat16).reshape(nb, L, head_dim)
    for j in range(nb):
        s = jax.lax.dot_general(qh[j], kh[j], tb,
                                preferred_element_type=jnp.float32)
        s = s - jnp.max(s, axis=-1, keepdims=True)
        e = jnp.exp(s)
        attn_ref[j] = e * (inv_heads / jnp.sum(e, axis=-1, keepdims=True))


def kernel(key, value, query, wq, wk, wv, wo):
    num_heads = 8
    B, Lk, Dk = key.shape
    _, Lv, Dv = value.shape
    _, Lq, _ = query.shape
    Dout = wo.shape[0]
    head_dim = Dk // num_heads
    lo = (num_heads - 1) * head_dim
    scale = head_dim ** (-0.5)

    wf2 = pl.pallas_call(
        _prep_kernel,
        out_shape=jax.ShapeDtypeStruct((Dout, Dv), jnp.bfloat16),
        compiler_params=pltpu.CompilerParams(
            vmem_limit_bytes=32 * _MIB),
    )(wo, wv)

    nb = 4 if B % 4 == 0 else 1
    grid_b = B // nb

    kern = functools.partial(_fused_kernel, lo=lo, head_dim=head_dim,
                             scale=scale, inv_heads=1.0 / num_heads, nb=nb)

    in_bytes = nb * 4 * (Lq * Dk + Lk * Dk + Lv * Dv)
    out_bytes = nb * 4 * (Lv * Dout + Lq * Lk)
    w_bytes = 4 * 2 * Dk * Dk + 2 * Dout * Dv
    vmem = 2 * (in_bytes + out_bytes) + w_bytes + 8 * nb * Lq * Lk * 4

    cost = pl.CostEstimate(
        flops=2 * B * (Lv * Dv * Dout + (Lq + Lk) * Dk * 128 + Lq * Lk * 128),
        transcendentals=B * Lq * Lk,
        bytes_accessed=grid_b * (in_bytes + out_bytes) + w_bytes)

    out, attn = pl.pallas_call(
        kern,
        out_shape=(jax.ShapeDtypeStruct((B, Lv, Dout), jnp.float32),
                   jax.ShapeDtypeStruct((B, Lq, Lk), jnp.float32)),
        grid=(grid_b,),
        in_specs=[
            pl.BlockSpec((nb, Lq, Dk), lambda b: (b, 0, 0)),
            pl.BlockSpec((nb, Lk, Dk), lambda b: (b, 0, 0)),
            pl.BlockSpec((nb, Lv, Dv), lambda b: (b, 0, 0)),
            pl.BlockSpec((Dk, Dk), lambda b: (0, 0)),
            pl.BlockSpec((Dk, Dk), lambda b: (0, 0)),
            pl.BlockSpec((Dout, Dv), lambda b: (0, 0)),
        ],
        out_specs=(pl.BlockSpec((nb, Lv, Dout), lambda b: (b, 0, 0)),
                   pl.BlockSpec((nb, Lq, Lk), lambda b: (b, 0, 0))),
        compiler_params=pltpu.CompilerParams(
            dimension_semantics=("parallel",),
            vmem_limit_bytes=int(min(max(vmem, 32 * _MIB), 64 * _MIB))),
        cost_estimate=cost,
    )(query, key, value, wq, wk, wf2)
    return out, attn
